# baseline (speedup 1.0000x reference)
import jax
import jax.numpy as jnp
import numpy as np
from jax import lax
from jax.experimental import pallas as pl
from jax.experimental.pallas import tpu as pltpu
from jax.experimental.pallas import tpu_sc as plsc

N = 10000
E = 320000
IN_F = 128
HID = 64
HEADS = 8
NCLS = 40

NC, NS, L = 2, 16, 16
NW = NC * NS
K = 128
EP = 327680
EPT1 = EP // NW
EPT2 = EP // NS
NCH1 = EPT1 // K
NCH2 = EPT2 // K
NPAD = 10112
NPT = NPAD // NS

NB = 400
GRID = N // NB

_f32 = jnp.float32
_i32 = jnp.int32


def _tc1_body(x_ref, w1_ref, c1_ref, z1t_ref, elr_ref, bnd_ref):
    i = pl.program_id(0)
    z = jnp.dot(x_ref[...], w1_ref[...], preferred_element_type=_f32)
    for p in range(HEADS // 2):
        z1t_ref[p, :, :] = z[:, 128 * p:128 * (p + 1)]
    elr = jnp.dot(z, c1_ref[...], preferred_element_type=_f32)
    elr_ref[...] = elr
    m = jnp.max(elr, axis=0, keepdims=True)

    @pl.when(i == 0)
    def _():
        bnd_ref[...] = m

    @pl.when(i > 0)
    def _():
        bnd_ref[...] = jnp.maximum(bnd_ref[...], m)

    @pl.when(i == pl.num_programs(0) - 1)
    def _():
        b = bnd_ref[...]
        s = b[:, 0:8] + b[:, 8:16]
        s = jnp.where(s >= 0, s, 0.2 * s)
        bnd_ref[...] = jnp.concatenate([s, s], axis=1)


def _tc1(x, W1, C1):
    return pl.pallas_call(
        _tc1_body,
        grid=(GRID,),
        in_specs=[
            pl.BlockSpec((NB, IN_F), lambda i: (i, 0)),
            pl.BlockSpec((IN_F, HEADS * HID), lambda i: (0, 0)),
            pl.BlockSpec((HEADS * HID, 16), lambda i: (0, 0)),
        ],
        out_specs=[
            pl.BlockSpec((HEADS // 2, NB, 2 * HID), lambda i: (0, i, 0)),
            pl.BlockSpec((NB, 16), lambda i: (i, 0)),
            pl.BlockSpec((1, 16), lambda i: (0, 0)),
        ],
        out_shape=[
            jax.ShapeDtypeStruct((HEADS // 2, N, 2 * HID), _f32),
            jax.ShapeDtypeStruct((N, 16), _f32),
            jax.ShapeDtypeStruct((1, 16), _f32),
        ],
    )(x, W1, C1)


def _tc2_body(o1_ref, den_ref, b1_ref, w2_ref, c2_ref,
              z2p_ref, elr2_ref, bnd2_ref):
    i = pl.program_id(0)
    den = den_ref[0] + den_ref[1]
    cols = []
    for h in range(HEADS):
        d = den[:, h:h + 1] + 1e-9
        q = h % 2
        v = o1_ref[h // 2][:, HID * q:HID * (q + 1)].astype(_f32) \
            / d + b1_ref[:, HID * h:HID * (h + 1)]
        v = jnp.where(v > 0, v, jnp.exp(jnp.minimum(v, 0.0)) - 1.0)
        cols.append(v)
    hb = jnp.concatenate(cols, axis=1)
    z2 = jnp.dot(hb, w2_ref[...], preferred_element_type=_f32)
    elr2 = jnp.dot(z2, c2_ref[...], preferred_element_type=_f32)
    z2p_ref[...] = jnp.concatenate(
        [z2, elr2[:, 0:2], jnp.zeros((NB, 6), _f32)], axis=1)
    elr2_ref[...] = elr2
    m = jnp.max(elr2, axis=0, keepdims=True)

    @pl.when(i == 0)
    def _():
        bnd2_ref[...] = m

    @pl.when(i > 0)
    def _():
        bnd2_ref[...] = jnp.maximum(bnd2_ref[...], m)

    @pl.when(i == pl.num_programs(0) - 1)
    def _():
        b = bnd2_ref[...]
        s = b[:, 0:1] + b[:, 1:2]
        s = jnp.where(s >= 0, s, 0.2 * s)
        bnd2_ref[...] = jnp.broadcast_to(s, (1, 16))


def _tc2(out1, den, b1, W2, C2):
    return pl.pallas_call(
        _tc2_body,
        grid=(GRID,),
        in_specs=[
            pl.BlockSpec((HEADS // 2, NB, 2 * HID), lambda i: (0, i, 0)),
            pl.BlockSpec((2, NB, 16), lambda i: (0, i, 0)),
            pl.BlockSpec((1, HEADS * HID), lambda i: (0, 0)),
            pl.BlockSpec((HEADS * HID, NCLS), lambda i: (0, 0)),
            pl.BlockSpec((NCLS, 16), lambda i: (0, 0)),
        ],
        out_specs=[
            pl.BlockSpec((NB, 48), lambda i: (i, 0)),
            pl.BlockSpec((NB, 16), lambda i: (i, 0)),
            pl.BlockSpec((1, 16), lambda i: (0, 0)),
        ],
        out_shape=[
            jax.ShapeDtypeStruct((N, 48), _f32),
            jax.ShapeDtypeStruct((N, 16), _f32),
            jax.ShapeDtypeStruct((1, 16), _f32),
        ],
    )(out1, den, b1, W2, C2)


def _tc3_body(a2_ref, b2_ref, out_ref):
    s = a2_ref[0] + a2_ref[1]
    d = s[:, 48:49] + 1e-9
    out_ref[...] = s[:, 0:NCLS] / d + b2_ref[...]


def _tc3(acc2, b2):
    return pl.pallas_call(
        _tc3_body,
        grid=(GRID,),
        in_specs=[
            pl.BlockSpec((2, NB, 64), lambda i: (0, i, 0)),
            pl.BlockSpec((1, NCLS), lambda i: (0, 0)),
        ],
        out_specs=pl.BlockSpec((NB, NCLS), lambda i: (i, 0)),
        out_shape=jax.ShapeDtypeStruct((N, NCLS), _f32),
    )(acc2, b2)


def _p1_body(elr_hbm, bnd_hbm, sd_hbm,
             ee_hbm, den_hbm,
             den_sh, zbuf, rec0, rec1, sidx, didx,
             es0, es1, ed0, ed1, eeb0, eeb1, bndv,
             lsem, gs0, gs1, gd0, gd1, ss0, ss1, ws0, ws1):
    cid = lax.axis_index("c")
    sid = lax.axis_index("s")
    wid = cid * NS + sid
    it16 = lax.iota(_i32, 16)
    recs = (rec0, rec1)
    ess = (es0, es1)
    eds = (ed0, ed1)
    eebs = (eeb0, eeb1)
    gss = (gs0, gs1)
    gds = (gd0, gd1)
    sss = (ss0, ss1)
    wss = (ws0, ws1)
    c0 = jnp.full((16,), 0, _i32)
    c1 = c0 + 1
    c8 = c0 + 8
    c9 = c0 + 9

    def _zr(r, _):
        zbuf[r, :] = jnp.zeros((16,), _f32)
        return 0

    lax.fori_loop(0, NPT, _zr, 0)
    pltpu.sync_copy(zbuf, den_sh.at[pl.ds(sid * NPT, NPT)])
    pltpu.sync_copy(bnd_hbm, bndv)
    plsc.subcore_barrier()

    base0 = wid * EPT1

    def issue_small(i, b):
        pltpu.async_copy(sd_hbm.at[pl.ds(base0 + i * K, K)], recs[b], lsem)

    def wait_small(i, b):
        pltpu.make_async_copy(sd_hbm.at[pl.ds(base0 + i * K, K)],
                              recs[b], lsem).wait()

    def stage_a(b, d):
        for g in range(8):
            rows = it16 + (16 * g)
            sl = pl.ds(16 * g, 16)
            sidx[b, sl] = plsc.load_gather(recs[b], [rows, c0])
            didx[d, sl] = plsc.load_gather(recs[b], [rows, c1])
        pltpu.async_copy(elr_hbm.at[sidx.at[b]], ess[b], gss[b])
        pltpu.async_copy(elr_hbm.at[didx.at[d]], eds[b], gds[b])

    def wait_gathers(b, d):
        pltpu.make_async_copy(elr_hbm.at[sidx.at[b]], ess[b], gss[b]).wait()
        pltpu.make_async_copy(elr_hbm.at[didx.at[d]], eds[b], gds[b]).wait()

    def issue_out(i, b, d):
        pltpu.async_copy(eebs[b], den_sh.at[didx.at[d]], sss[b], add=True)
        pltpu.async_copy(eebs[b], ee_hbm.at[pl.ds(base0 + i * K, K)], wss[b])

    def wait_out(i, b, d):
        pltpu.make_async_copy(eebs[b], den_sh.at[didx.at[d]], sss[b]).wait()
        pltpu.make_async_copy(eebs[b], ee_hbm.at[pl.ds(base0 + i * K, K)],
                              wss[b]).wait()

    issue_small(0, 0)
    wait_small(0, 0)
    stage_a(0, 0)
    issue_small(1, 1)

    def outer(io, _):
        for u in range(4):
            i = 4 * io + u
            b = u % 2
            d = u
            b1 = (u + 1) % 2
            d1 = (u + 1) % 4

            @pl.when(i >= 2)
            def _():
                wait_out(i - 2, b, (u + 2) % 4)

            @pl.when(i < NCH1 - 1)
            def _():
                wait_small(i + 1, b1)
                stage_a(b1, d1)

            @pl.when(i < NCH1 - 2)
            def _():
                issue_small(i + 2, b)

            wait_gathers(b, d)
            _compute_p1(i, b, d, base0, bndv, it16, ess, eds, eebs,
                        sidx, didx)
            issue_out(i, b, d)
        return 0

    lax.fori_loop(0, NCH1 // 4, outer, 0)
    wait_out(NCH1 - 2, 0, 2)
    wait_out(NCH1 - 1, 1, 3)
    plsc.subcore_barrier()
    pltpu.sync_copy(den_sh.at[pl.ds(sid * NPT, NPT)],
                    den_hbm.at[cid, pl.ds(sid * NPT, NPT)])


def _compute_p1(i, b, d, base0, bndv, it16, ess, eds, eebs, sidx, didx):
    base = base0 + i * K
    c0 = jnp.full((16,), 0, _i32)
    c8 = c0 + 8
    c9 = c0 + 9
    brow = bndv[0]
    for g in range(8):
        rows = it16 + (16 * g)
        sl = pl.ds(16 * g, 16)
        valid = (base + 16 * g + it16) < E
        for h in range(HEADS):
            el = plsc.load_gather(ess[b], [rows, c0 + h])
            er = plsc.load_gather(eds[b], [rows, c8 + h])
            ev = el + er
            ev = jnp.where(ev >= 0, ev, 0.2 * ev)
            ee = jnp.exp(ev - brow[h])
            ee = jnp.where(valid, ee, 0.0)
            plsc.store_scatter(eebs[b], [rows, c0 + h], ee)
        plsc.store_scatter(eebs[b], [rows, c8],
                           plsc.bitcast(sidx[b, sl], _f32))
        plsc.store_scatter(eebs[b], [rows, c9],
                           plsc.bitcast(didx[d, sl], _f32))


def _p1(elr, bnd, srcdst):
    mesh = plsc.VectorSubcoreMesh(core_axis_name="c", subcore_axis_name="s")
    f = pl.kernel(
        _p1_body,
        out_type=(jax.ShapeDtypeStruct((EP, 16), _f32),
                  jax.ShapeDtypeStruct((2, NPAD, 16), _f32)),
        mesh=mesh,
        compiler_params=pltpu.CompilerParams(needs_layout_passes=False, use_tc_tiling_on_sc=False),
        scratch_types=[
            pltpu.VMEM_SHARED((NPAD, 16), _f32),
            pltpu.VMEM((NPT, 16), _f32),
            pltpu.VMEM((K, 2), _i32),
            pltpu.VMEM((K, 2), _i32),
            pltpu.VMEM((2, K), _i32),
            pltpu.VMEM((4, K), _i32),
            pltpu.VMEM((K, 16), _f32),
            pltpu.VMEM((K, 16), _f32),
            pltpu.VMEM((K, 16), _f32),
            pltpu.VMEM((K, 16), _f32),
            pltpu.VMEM((K, 16), _f32),
            pltpu.VMEM((K, 16), _f32),
            pltpu.VMEM((1, 16), _f32),
        ] + [pltpu.SemaphoreType.DMA] * 9,
    )
    return f(elr, bnd, srcdst)


K2 = 128
NCH2B = EPT2 // K2
_bf16 = jnp.bfloat16


def _p2_body(z1t_hbm, ee_hbm, out1_hbm,
             acc_sh, eeA0, eeA1, idxb, dbuf, wb0, wb1, zr, stb,
             lsem, gsem0, gsem1, ssem0, ssem1):
    cid = lax.axis_index("c")
    sid = lax.axis_index("s")
    it16 = lax.iota(_i32, 16)
    eeAs = (eeA0, eeA1)
    gsems = (gsem0, gsem1)
    ssems = (ssem0, ssem1)
    NG = K2 // 16

    base00 = sid * EPT2
    c8 = jnp.full((16,), 8, _i32)
    c9 = jnp.full((16,), 9, _i32)

    def _z(r, _):
        for j in range(4):
            stb[0, r, pl.ds(32 * j, 32)] = jnp.zeros((32,), _bf16)
        return 0

    for pp in range(2):
        p = cid * 2 + pp
        pN = p * N
        pNo = p * NPAD
        lax.fori_loop(0, K2, _z, 0)
        for t in range(4):
            pltpu.sync_copy(
                stb.at[0], acc_sh.at[pl.ds(sid * NPT + t * K2, K2)])
        pltpu.sync_copy(stb.at[0, pl.ds(0, NPT - 4 * K2)],
                        acc_sh.at[pl.ds(sid * NPT + 4 * K2, NPT - 4 * K2)])
        plsc.subcore_barrier()

        c2p = jnp.full((16,), 0, _i32) + 2 * p
        c2p1 = c2p + 1

        def issue_small(i, sb):
            base = base00 + i * K2
            pltpu.async_copy(ee_hbm.at[pl.ds(base, K2)], eeAs[sb], lsem)

        def wait_small(i, sb):
            base = base00 + i * K2
            pltpu.make_async_copy(ee_hbm.at[pl.ds(base, K2)], eeAs[sb],
                                  lsem).wait()

        def stage_a(b, d):
            for g in range(NG):
                rows = it16 + (16 * g)
                sl = pl.ds(16 * g, 16)
                wb0[b, sl] = plsc.load_gather(eeAs[b], [rows, c2p])
                wb1[b, sl] = plsc.load_gather(eeAs[b], [rows, c2p1])
                sv = plsc.bitcast(
                    plsc.load_gather(eeAs[b], [rows, c8]), _i32)
                idxb[b, sl] = sv + pN
                dbuf[d, sl] = plsc.bitcast(
                    plsc.load_gather(eeAs[b], [rows, c9]), _i32)
            pltpu.async_copy(z1t_hbm.at[idxb.at[b]], zr.at[b], gsems[b])

        def wait_gather(b):
            pltpu.make_async_copy(z1t_hbm.at[idxb.at[b]], zr.at[b],
                                  gsems[b]).wait()

        def issue_scatter(b, d):
            pltpu.async_copy(stb.at[b], acc_sh.at[dbuf.at[d]],
                             ssems[b], add=True)

        def wait_scatter(b, d):
            pltpu.make_async_copy(stb.at[b], acc_sh.at[dbuf.at[d]],
                                  ssems[b]).wait()

        def compute(b):
            cb = jnp.full((16,), b, _i32)

            def mbody(g, _):
                ge = jnp.full((16,), 16, _i32) * g

                for l in range(16):
                    e = 16 * g + l
                    il = ge + l
                    w0 = plsc.load_gather(wb0, [cb, il])
                    w1 = plsc.load_gather(wb1, [cb, il])
                    pr = []
                    for j in range(4):
                        pr.append(zr[b, e, pl.ds(16 * j, 16)] * w0)
                    for j in range(4, 8):
                        pr.append(zr[b, e, pl.ds(16 * j, 16)] * w1)
                    for jj in range(4):
                        stb[b, e, pl.ds(32 * jj, 32)] = plsc.pack(
                            pr[2 * jj], pr[2 * jj + 1],
                            format=plsc.PackFormat.INTERLEAVED)
                return 0

            lax.fori_loop(0, NG, mbody, 0)

        issue_small(0, 0)
        wait_small(0, 0)
        stage_a(0, 0)
        issue_small(1, 1)

        def outer(io, _):
            for u in range(4):
                i = 4 * io + u
                b = u % 2
                d = u
                b1 = (u + 1) % 2
                d1 = (u + 1) % 4

                @pl.when(i >= 2)
                def _():
                    wait_scatter(b, (u + 2) % 4)

                @pl.when(i < NCH2B - 1)
                def _():
                    wait_small(i + 1, b1)
                    stage_a(b1, d1)

                @pl.when(i < NCH2B - 2)
                def _():
                    issue_small(i + 2, b)

                wait_gather(b)
                compute(b)
                issue_scatter(b, d)
            return 0

        lax.fori_loop(0, NCH2B // 4, outer, 0)
        wait_scatter(0, 2)
        wait_scatter(1, 3)
        plsc.subcore_barrier()
        pltpu.sync_copy(acc_sh.at[pl.ds(sid * NPT, NPT)],
                        out1_hbm.at[pl.ds(pNo + sid * NPT, NPT)])
        plsc.subcore_barrier()


def _p2(z1t, ee):
    mesh = plsc.VectorSubcoreMesh(core_axis_name="c", subcore_axis_name="s")
    f = pl.kernel(
        _p2_body,
        out_type=jax.ShapeDtypeStruct((4 * NPAD, 2 * HID), _bf16),
        mesh=mesh,
        compiler_params=pltpu.CompilerParams(needs_layout_passes=False, use_tc_tiling_on_sc=False),
        scratch_types=[
            pltpu.VMEM_SHARED((NPAD, 2 * HID), _bf16),
            pltpu.VMEM((K2, 16), _f32),
            pltpu.VMEM((K2, 16), _f32),
            pltpu.VMEM((2, K2), _i32),
            pltpu.VMEM((4, K2), _i32),
            pltpu.VMEM((2, K2), _f32),
            pltpu.VMEM((2, K2), _f32),
            pltpu.VMEM((2, K2, 2 * HID), _f32),
            pltpu.VMEM((2, K2, 2 * HID), _bf16),
            pltpu.SemaphoreType.DMA,
            pltpu.SemaphoreType.DMA,
            pltpu.SemaphoreType.DMA,
            pltpu.SemaphoreType.DMA,
            pltpu.SemaphoreType.DMA,
        ],
    )
    return f(z1t, ee)


def _p3_body(z2p_hbm, bnd2_hbm, sd_hbm, acc2_hbm,
             acc_sh, zbuf, rec0, rec1, sidx, didx,
             zr0, zr1, ed0, ed1, stb0, stb1, wbuf, bndv,
             lsem, gs0, gs1, gd0, gd1, ss0, ss1):
    cid = lax.axis_index("c")
    sid = lax.axis_index("s")
    wid = cid * NS + sid
    it16 = lax.iota(_i32, 16)
    unit = jnp.where(it16 == 0, 1.0, 0.0).astype(_f32)
    recs = (rec0, rec1)
    zrs = (zr0, zr1)
    eds = (ed0, ed1)
    stbs = (stb0, stb1)
    gss = (gs0, gs1)
    gds = (gd0, gd1)
    sss = (ss0, ss1)
    c0 = jnp.full((16,), 0, _i32)
    c1 = c0 + 1
    c40 = c0 + 40
    c41 = c0 + 41

    def _z(r, _):
        for j in range(4):
            zbuf[r, pl.ds(16 * j, 16)] = jnp.zeros((16,), _f32)
        return 0

    lax.fori_loop(0, NPT // 4, _z, 0)
    for t in range(4):
        pltpu.sync_copy(zbuf,
                        acc_sh.at[pl.ds(sid * NPT + t * (NPT // 4),
                                        NPT // 4)])
    pltpu.sync_copy(bnd2_hbm, bndv)
    plsc.subcore_barrier()

    base0 = wid * EPT1

    def issue_small(i, b):
        pltpu.async_copy(sd_hbm.at[pl.ds(base0 + i * K, K)], recs[b], lsem)

    def wait_small(i, b):
        pltpu.make_async_copy(sd_hbm.at[pl.ds(base0 + i * K, K)],
                              recs[b], lsem).wait()

    def stage_a(b, d):
        for g in range(8):
            rows = it16 + (16 * g)
            sl = pl.ds(16 * g, 16)
            sidx[b, sl] = plsc.load_gather(recs[b], [rows, c0])
            didx[d, sl] = plsc.load_gather(recs[b], [rows, c1])
        pltpu.async_copy(z2p_hbm.at[sidx.at[b]], zrs[b], gss[b])
        pltpu.async_copy(z2p_hbm.at[didx.at[d]], eds[b], gds[b])

    def wait_gathers(b, d):
        pltpu.make_async_copy(z2p_hbm.at[sidx.at[b]], zrs[b], gss[b]).wait()
        pltpu.make_async_copy(z2p_hbm.at[didx.at[d]], eds[b], gds[b]).wait()

    def compute(i, b):
        base = base0 + i * K
        bnd = bndv[0]
        for g in range(8):
            rows = it16 + (16 * g)
            el = plsc.load_gather(zrs[b], [rows, c40])
            er = plsc.load_gather(eds[b], [rows, c41])
            ev = el + er
            ev = jnp.where(ev >= 0, ev, 0.2 * ev)
            ee = jnp.exp(ev - bnd)
            valid = (base + 16 * g + it16) < E
            wbuf[pl.ds(16 * g, 16)] = jnp.where(valid, ee, 0.0)

        def mbody(g, _):
            ge = jnp.full((16,), 16, _i32) * g
            for l in range(16):
                e = 16 * g + l
                w = plsc.load_gather(wbuf, [ge + l])
                for j in range(3):
                    sj = pl.ds(16 * j, 16)
                    stbs[b][e, sj] = zrs[b][e, sj] * w
                stbs[b][e, pl.ds(48, 16)] = unit * w
            return 0

        lax.fori_loop(0, K // 16, mbody, 0)

    def issue_scatter(b, d):
        pltpu.async_copy(stbs[b], acc_sh.at[didx.at[d]], sss[b], add=True)

    def wait_scatter(b, d):
        pltpu.make_async_copy(stbs[b], acc_sh.at[didx.at[d]], sss[b]).wait()

    issue_small(0, 0)
    wait_small(0, 0)
    stage_a(0, 0)
    issue_small(1, 1)

    def outer(io, _):
        for u in range(4):
            i = 4 * io + u
            b = u % 2
            d = u
            b1 = (u + 1) % 2
            d1 = (u + 1) % 4

            @pl.when(i >= 2)
            def _():
                wait_scatter(b, (u + 2) % 4)

            @pl.when(i < NCH1 - 1)
            def _():
                wait_small(i + 1, b1)
                stage_a(b1, d1)

            @pl.when(i < NCH1 - 2)
            def _():
                issue_small(i + 2, b)

            wait_gathers(b, d)
            compute(i, b)
            issue_scatter(b, d)
        return 0

    lax.fori_loop(0, NCH1 // 4, outer, 0)
    wait_scatter(0, 2)
    wait_scatter(1, 3)
    plsc.subcore_barrier()
    pltpu.sync_copy(acc_sh.at[pl.ds(sid * NPT, NPT)],
                    acc2_hbm.at[cid, pl.ds(sid * NPT, NPT)])


def _p3(z2p, bnd2, srcdst):
    mesh = plsc.VectorSubcoreMesh(core_axis_name="c", subcore_axis_name="s")
    f = pl.kernel(
        _p3_body,
        out_type=jax.ShapeDtypeStruct((2, NPAD, 64), _f32),
        mesh=mesh,
        compiler_params=pltpu.CompilerParams(needs_layout_passes=False, use_tc_tiling_on_sc=False),
        scratch_types=[
            pltpu.VMEM_SHARED((NPAD, 64), _f32),
            pltpu.VMEM((NPT // 4, 64), _f32),
            pltpu.VMEM((K, 2), _i32),
            pltpu.VMEM((K, 2), _i32),
            pltpu.VMEM((2, K), _i32),
            pltpu.VMEM((4, K), _i32),
            pltpu.VMEM((K, 48), _f32),
            pltpu.VMEM((K, 48), _f32),
            pltpu.VMEM((K, 48), _f32),
            pltpu.VMEM((K, 48), _f32),
            pltpu.VMEM((K, 64), _f32),
            pltpu.VMEM((K, 64), _f32),
            pltpu.VMEM((K,), _f32),
            pltpu.VMEM((1, 16), _f32),
        ] + [pltpu.SemaphoreType.DMA] * 7,
    )
    return f(z2p, bnd2, srcdst)


def kernel(x, edge_index, W1, a_l1, a_r1, b1, W2, a_l2, a_r2, b2):
    src = edge_index[0]
    dst = edge_index[1]
    pad = jnp.zeros((EP - E,), _i32)
    srcp = jnp.concatenate([src, pad])
    dstp = jnp.concatenate([dst, pad])
    srcdst = jnp.stack([srcp, dstp], axis=1)

    perm64 = np.array([32 * (l // 32) + 16 * (l % 2) + (l % 32) // 2
                       for l in range(64)])
    perm512 = np.concatenate([h * 64 + perm64 for h in range(HEADS)])
    b1p = b1[perm512]
    W2p = W2[perm512, :]

    eye8 = jnp.eye(HEADS, dtype=_f32)
    C_l = (a_l1[:, :, None] * eye8[:, None, :]).reshape(HEADS * HID, HEADS)
    C_r = (a_r1[:, :, None] * eye8[:, None, :]).reshape(HEADS * HID, HEADS)
    C1 = jnp.concatenate([C_l, C_r], axis=1)
    C2 = jnp.zeros((NCLS, 16), _f32).at[:, 0].set(a_l2[0]).at[:, 1].set(a_r2[0])

    z1t, elr, bnd = _tc1(x, W1, C1)
    z1t = z1t.reshape(4 * N, 2 * HID)
    ee, den = _p1(elr, bnd, srcdst)
    out1 = _p2(z1t, ee)
    out1 = out1.reshape(4, NPAD, 2 * HID)
    z2p, elr2, bnd2 = _tc2(out1, den, b1p.reshape(1, -1), W2p, C2)
    acc2 = _p3(z2p, bnd2, srcdst)
    return _tc3(acc2, b2.reshape(1, -1))


# speedup vs baseline: 1.1122x; 1.1122x over previous
import jax
import jax.numpy as jnp
import numpy as np
from jax import lax
from jax.experimental import pallas as pl
from jax.experimental.pallas import tpu as pltpu
from jax.experimental.pallas import tpu_sc as plsc

N = 10000
E = 320000
IN_F = 128
HID = 64
HEADS = 8
NCLS = 40

NC, NS, L = 2, 16, 16
NW = NC * NS
K = 128
EP = 327680
EPT1 = EP // NW
EPT2 = EP // NS
NCH1 = EPT1 // K
NCH2 = EPT2 // K
NPAD = 10112
NPT = NPAD // NS

NB = 400
GRID = N // NB

_f32 = jnp.float32
_i32 = jnp.int32


def _tc1_body(x_ref, w1_ref, c1_ref, z1t_ref, elr_ref, bnd_ref):
    i = pl.program_id(0)
    z = jnp.dot(x_ref[...], w1_ref[...], preferred_element_type=_f32)
    for p in range(HEADS // 2):
        z1t_ref[p, :, :] = z[:, 128 * p:128 * (p + 1)].astype(jnp.bfloat16)
    elr = jnp.dot(z, c1_ref[...], preferred_element_type=_f32)
    elr_ref[...] = elr
    m = jnp.max(elr, axis=0, keepdims=True)

    @pl.when(i == 0)
    def _():
        bnd_ref[...] = m

    @pl.when(i > 0)
    def _():
        bnd_ref[...] = jnp.maximum(bnd_ref[...], m)

    @pl.when(i == pl.num_programs(0) - 1)
    def _():
        b = bnd_ref[...]
        s = b[:, 0:8] + b[:, 8:16]
        s = jnp.where(s >= 0, s, 0.2 * s)
        bnd_ref[...] = jnp.concatenate([s, s], axis=1)


def _tc1(x, W1, C1):
    return pl.pallas_call(
        _tc1_body,
        grid=(GRID,),
        in_specs=[
            pl.BlockSpec((NB, IN_F), lambda i: (i, 0)),
            pl.BlockSpec((IN_F, HEADS * HID), lambda i: (0, 0)),
            pl.BlockSpec((HEADS * HID, 16), lambda i: (0, 0)),
        ],
        out_specs=[
            pl.BlockSpec((HEADS // 2, NB, 2 * HID), lambda i: (0, i, 0)),
            pl.BlockSpec((NB, 16), lambda i: (i, 0)),
            pl.BlockSpec((1, 16), lambda i: (0, 0)),
        ],
        out_shape=[
            jax.ShapeDtypeStruct((HEADS // 2, N, 2 * HID), jnp.bfloat16),
            jax.ShapeDtypeStruct((N, 16), _f32),
            jax.ShapeDtypeStruct((1, 16), _f32),
        ],
    )(x, W1, C1)


def _tc2_body(o1_ref, den_ref, b1_ref, w2_ref, c2_ref,
              z2p_ref, elr2_ref, bnd2_ref):
    i = pl.program_id(0)
    den = den_ref[0] + den_ref[1]
    cols = []
    for h in range(HEADS):
        d = den[:, h:h + 1] + 1e-9
        q = h % 2
        v = o1_ref[h // 2][:, HID * q:HID * (q + 1)].astype(_f32) \
            / d + b1_ref[:, HID * h:HID * (h + 1)]
        v = jnp.where(v > 0, v, jnp.exp(jnp.minimum(v, 0.0)) - 1.0)
        cols.append(v)
    hb = jnp.concatenate(cols, axis=1)
    z2 = jnp.dot(hb, w2_ref[...], preferred_element_type=_f32)
    elr2 = jnp.dot(z2, c2_ref[...], preferred_element_type=_f32)
    z2p_ref[...] = jnp.concatenate(
        [z2, elr2[:, 0:2], jnp.zeros((NB, 6), _f32)], axis=1)
    elr2_ref[...] = elr2
    m = jnp.max(elr2, axis=0, keepdims=True)

    @pl.when(i == 0)
    def _():
        bnd2_ref[...] = m

    @pl.when(i > 0)
    def _():
        bnd2_ref[...] = jnp.maximum(bnd2_ref[...], m)

    @pl.when(i == pl.num_programs(0) - 1)
    def _():
        b = bnd2_ref[...]
        s = b[:, 0:1] + b[:, 1:2]
        s = jnp.where(s >= 0, s, 0.2 * s)
        bnd2_ref[...] = jnp.broadcast_to(s, (1, 16))


def _tc2(out1, den, b1, W2, C2):
    return pl.pallas_call(
        _tc2_body,
        grid=(GRID,),
        in_specs=[
            pl.BlockSpec((HEADS // 2, NB, 2 * HID), lambda i: (0, i, 0)),
            pl.BlockSpec((2, NB, 16), lambda i: (0, i, 0)),
            pl.BlockSpec((1, HEADS * HID), lambda i: (0, 0)),
            pl.BlockSpec((HEADS * HID, NCLS), lambda i: (0, 0)),
            pl.BlockSpec((NCLS, 16), lambda i: (0, 0)),
        ],
        out_specs=[
            pl.BlockSpec((NB, 48), lambda i: (i, 0)),
            pl.BlockSpec((NB, 16), lambda i: (i, 0)),
            pl.BlockSpec((1, 16), lambda i: (0, 0)),
        ],
        out_shape=[
            jax.ShapeDtypeStruct((N, 48), _f32),
            jax.ShapeDtypeStruct((N, 16), _f32),
            jax.ShapeDtypeStruct((1, 16), _f32),
        ],
    )(out1, den, b1, W2, C2)


def _tc3_body(a2_ref, b2_ref, out_ref):
    s = a2_ref[0] + a2_ref[1]
    d = s[:, 48:49] + 1e-9
    out_ref[...] = s[:, 0:NCLS] / d + b2_ref[...]


def _tc3(acc2, b2):
    return pl.pallas_call(
        _tc3_body,
        grid=(GRID,),
        in_specs=[
            pl.BlockSpec((2, NB, 64), lambda i: (0, i, 0)),
            pl.BlockSpec((1, NCLS), lambda i: (0, 0)),
        ],
        out_specs=pl.BlockSpec((NB, NCLS), lambda i: (i, 0)),
        out_shape=jax.ShapeDtypeStruct((N, NCLS), _f32),
    )(acc2, b2)


def _p1_body(elr_hbm, bnd_hbm, sd_hbm,
             ee_hbm, den_hbm,
             den_sh, zbuf, rec0, rec1, sidx, didx,
             es0, es1, ed0, ed1, eeb0, eeb1, bndv,
             lsem, gs0, gs1, gd0, gd1, ss0, ss1, ws0, ws1):
    cid = lax.axis_index("c")
    sid = lax.axis_index("s")
    wid = cid * NS + sid
    it16 = lax.iota(_i32, 16)
    recs = (rec0, rec1)
    ess = (es0, es1)
    eds = (ed0, ed1)
    eebs = (eeb0, eeb1)
    gss = (gs0, gs1)
    gds = (gd0, gd1)
    sss = (ss0, ss1)
    wss = (ws0, ws1)
    c0 = jnp.full((16,), 0, _i32)
    c1 = c0 + 1
    c8 = c0 + 8
    c9 = c0 + 9

    def _zr(r, _):
        zbuf[r, :] = jnp.zeros((16,), _f32)
        return 0

    lax.fori_loop(0, NPT, _zr, 0)
    pltpu.sync_copy(zbuf, den_sh.at[pl.ds(sid * NPT, NPT)])
    pltpu.sync_copy(bnd_hbm, bndv)
    plsc.subcore_barrier()

    base0 = wid * EPT1

    def issue_small(i, b):
        pltpu.async_copy(sd_hbm.at[pl.ds(base0 + i * K, K)], recs[b], lsem)

    def wait_small(i, b):
        pltpu.make_async_copy(sd_hbm.at[pl.ds(base0 + i * K, K)],
                              recs[b], lsem).wait()

    def stage_a(b, d):
        for g in range(8):
            rows = it16 + (16 * g)
            sl = pl.ds(16 * g, 16)
            sidx[b, sl] = plsc.load_gather(recs[b], [rows, c0])
            didx[d, sl] = plsc.load_gather(recs[b], [rows, c1])
        pltpu.async_copy(elr_hbm.at[sidx.at[b]], ess[b], gss[b])
        pltpu.async_copy(elr_hbm.at[didx.at[d]], eds[b], gds[b])

    def wait_gathers(b, d):
        pltpu.make_async_copy(elr_hbm.at[sidx.at[b]], ess[b], gss[b]).wait()
        pltpu.make_async_copy(elr_hbm.at[didx.at[d]], eds[b], gds[b]).wait()

    def issue_out(i, b, d):
        pltpu.async_copy(eebs[b], den_sh.at[didx.at[d]], sss[b], add=True)
        pltpu.async_copy(eebs[b], ee_hbm.at[pl.ds(base0 + i * K, K)], wss[b])

    def wait_out(i, b, d):
        pltpu.make_async_copy(eebs[b], den_sh.at[didx.at[d]], sss[b]).wait()
        pltpu.make_async_copy(eebs[b], ee_hbm.at[pl.ds(base0 + i * K, K)],
                              wss[b]).wait()

    issue_small(0, 0)
    wait_small(0, 0)
    stage_a(0, 0)
    issue_small(1, 1)

    def outer(io, _):
        for u in range(4):
            i = 4 * io + u
            b = u % 2
            d = u
            b1 = (u + 1) % 2
            d1 = (u + 1) % 4

            @pl.when(i >= 2)
            def _():
                wait_out(i - 2, b, (u + 2) % 4)

            @pl.when(i < NCH1 - 1)
            def _():
                wait_small(i + 1, b1)
                stage_a(b1, d1)

            @pl.when(i < NCH1 - 2)
            def _():
                issue_small(i + 2, b)

            wait_gathers(b, d)
            _compute_p1(i, b, d, base0, bndv, it16, ess, eds, eebs,
                        sidx, didx)
            issue_out(i, b, d)
        return 0

    lax.fori_loop(0, NCH1 // 4, outer, 0)
    wait_out(NCH1 - 2, 0, 2)
    wait_out(NCH1 - 1, 1, 3)
    plsc.subcore_barrier()
    pltpu.sync_copy(den_sh.at[pl.ds(sid * NPT, NPT)],
                    den_hbm.at[cid, pl.ds(sid * NPT, NPT)])


def _compute_p1(i, b, d, base0, bndv, it16, ess, eds, eebs, sidx, didx):
    base = base0 + i * K
    c0 = jnp.full((16,), 0, _i32)
    c8 = c0 + 8
    c9 = c0 + 9
    brow = bndv[0]
    for g in range(8):
        rows = it16 + (16 * g)
        sl = pl.ds(16 * g, 16)
        valid = (base + 16 * g + it16) < E
        for h in range(HEADS):
            el = plsc.load_gather(ess[b], [rows, c0 + h])
            er = plsc.load_gather(eds[b], [rows, c8 + h])
            ev = el + er
            ev = jnp.where(ev >= 0, ev, 0.2 * ev)
            ee = jnp.exp(ev - brow[h])
            ee = jnp.where(valid, ee, 0.0)
            plsc.store_scatter(eebs[b], [rows, c0 + h], ee)
        plsc.store_scatter(eebs[b], [rows, c8],
                           plsc.bitcast(sidx[b, sl], _f32))
        plsc.store_scatter(eebs[b], [rows, c9],
                           plsc.bitcast(didx[d, sl], _f32))


def _p1(elr, bnd, srcdst):
    mesh = plsc.VectorSubcoreMesh(core_axis_name="c", subcore_axis_name="s")
    f = pl.kernel(
        _p1_body,
        out_type=(jax.ShapeDtypeStruct((EP, 16), _f32),
                  jax.ShapeDtypeStruct((2, NPAD, 16), _f32)),
        mesh=mesh,
        compiler_params=pltpu.CompilerParams(needs_layout_passes=False, use_tc_tiling_on_sc=False),
        scratch_types=[
            pltpu.VMEM_SHARED((NPAD, 16), _f32),
            pltpu.VMEM((NPT, 16), _f32),
            pltpu.VMEM((K, 2), _i32),
            pltpu.VMEM((K, 2), _i32),
            pltpu.VMEM((2, K), _i32),
            pltpu.VMEM((4, K), _i32),
            pltpu.VMEM((K, 16), _f32),
            pltpu.VMEM((K, 16), _f32),
            pltpu.VMEM((K, 16), _f32),
            pltpu.VMEM((K, 16), _f32),
            pltpu.VMEM((K, 16), _f32),
            pltpu.VMEM((K, 16), _f32),
            pltpu.VMEM((1, 16), _f32),
        ] + [pltpu.SemaphoreType.DMA] * 9,
    )
    return f(elr, bnd, srcdst)


K2 = 128
NCH2B = EPT2 // K2
_bf16 = jnp.bfloat16


def _p2_body(z1t_hbm, ee_hbm, out1_hbm,
             acc_sh, eeA0, eeA1, idxb, dbuf, wb0, wb1, zr, stb,
             lsem, gsem0, gsem1, ssem0, ssem1):
    cid = lax.axis_index("c")
    sid = lax.axis_index("s")
    it16 = lax.iota(_i32, 16)
    eeAs = (eeA0, eeA1)
    gsems = (gsem0, gsem1)
    ssems = (ssem0, ssem1)
    NG = K2 // 16

    base00 = sid * EPT2
    c8 = jnp.full((16,), 8, _i32)
    c9 = jnp.full((16,), 9, _i32)

    def _z(r, _):
        for j in range(4):
            stb[0, r, pl.ds(32 * j, 32)] = jnp.zeros((32,), _bf16)
        return 0

    for pp in range(2):
        p = cid * 2 + pp
        pN = p * N
        pNo = p * NPAD
        lax.fori_loop(0, K2, _z, 0)
        for t in range(4):
            pltpu.sync_copy(
                stb.at[0], acc_sh.at[pl.ds(sid * NPT + t * K2, K2)])
        pltpu.sync_copy(stb.at[0, pl.ds(0, NPT - 4 * K2)],
                        acc_sh.at[pl.ds(sid * NPT + 4 * K2, NPT - 4 * K2)])
        plsc.subcore_barrier()

        c2p = jnp.full((16,), 0, _i32) + 2 * p
        c2p1 = c2p + 1

        def issue_small(i, sb):
            base = base00 + i * K2
            pltpu.async_copy(ee_hbm.at[pl.ds(base, K2)], eeAs[sb], lsem)

        def wait_small(i, sb):
            base = base00 + i * K2
            pltpu.make_async_copy(ee_hbm.at[pl.ds(base, K2)], eeAs[sb],
                                  lsem).wait()

        def stage_a(b, d):
            for g in range(NG):
                rows = it16 + (16 * g)
                sl = pl.ds(16 * g, 16)
                wb0[b, sl] = plsc.load_gather(eeAs[b], [rows, c2p])
                wb1[b, sl] = plsc.load_gather(eeAs[b], [rows, c2p1])
                sv = plsc.bitcast(
                    plsc.load_gather(eeAs[b], [rows, c8]), _i32)
                idxb[b, sl] = sv + pN
                dbuf[d, sl] = plsc.bitcast(
                    plsc.load_gather(eeAs[b], [rows, c9]), _i32)
            pltpu.async_copy(z1t_hbm.at[idxb.at[b]], zr.at[b], gsems[b])

        def wait_gather(b):
            pltpu.make_async_copy(z1t_hbm.at[idxb.at[b]], zr.at[b],
                                  gsems[b]).wait()

        def issue_scatter(b, d):
            pltpu.async_copy(stb.at[b], acc_sh.at[dbuf.at[d]],
                             ssems[b], add=True)

        def wait_scatter(b, d):
            pltpu.make_async_copy(stb.at[b], acc_sh.at[dbuf.at[d]],
                                  ssems[b]).wait()

        def compute(b):
            cb = jnp.full((16,), b, _i32)

            def mbody(g, _):
                ge = jnp.full((16,), 16, _i32) * g

                for l in range(16):
                    e = 16 * g + l
                    il = ge + l
                    w0 = plsc.load_gather(wb0, [cb, il])
                    w1 = plsc.load_gather(wb1, [cb, il])
                    for jj in range(4):
                        zv = zr[b, e, pl.ds(32 * jj, 32)]
                        za, zb = plsc.unpack(
                            zv, format=plsc.PackFormat.INTERLEAVED)
                        w = w0 if jj < 2 else w1
                        stb[b, e, pl.ds(32 * jj, 32)] = plsc.pack(
                            za * w, zb * w,
                            format=plsc.PackFormat.INTERLEAVED)
                return 0

            lax.fori_loop(0, NG, mbody, 0)

        issue_small(0, 0)
        wait_small(0, 0)
        stage_a(0, 0)
        issue_small(1, 1)

        def outer(io, _):
            for u in range(4):
                i = 4 * io + u
                b = u % 2
                d = u
                b1 = (u + 1) % 2
                d1 = (u + 1) % 4

                @pl.when(i >= 2)
                def _():
                    wait_scatter(b, (u + 2) % 4)

                @pl.when(i < NCH2B - 1)
                def _():
                    wait_small(i + 1, b1)
                    stage_a(b1, d1)

                @pl.when(i < NCH2B - 2)
                def _():
                    issue_small(i + 2, b)

                wait_gather(b)
                compute(b)
                issue_scatter(b, d)
            return 0

        lax.fori_loop(0, NCH2B // 4, outer, 0)
        wait_scatter(0, 2)
        wait_scatter(1, 3)
        plsc.subcore_barrier()
        pltpu.sync_copy(acc_sh.at[pl.ds(sid * NPT, NPT)],
                        out1_hbm.at[pl.ds(pNo + sid * NPT, NPT)])
        plsc.subcore_barrier()


def _p2(z1t, ee):
    mesh = plsc.VectorSubcoreMesh(core_axis_name="c", subcore_axis_name="s")
    f = pl.kernel(
        _p2_body,
        out_type=jax.ShapeDtypeStruct((4 * NPAD, 2 * HID), _bf16),
        mesh=mesh,
        compiler_params=pltpu.CompilerParams(needs_layout_passes=False, use_tc_tiling_on_sc=False),
        scratch_types=[
            pltpu.VMEM_SHARED((NPAD, 2 * HID), _bf16),
            pltpu.VMEM((K2, 16), _f32),
            pltpu.VMEM((K2, 16), _f32),
            pltpu.VMEM((2, K2), _i32),
            pltpu.VMEM((4, K2), _i32),
            pltpu.VMEM((2, K2), _f32),
            pltpu.VMEM((2, K2), _f32),
            pltpu.VMEM((2, K2, 2 * HID), _bf16),
            pltpu.VMEM((2, K2, 2 * HID), _bf16),
            pltpu.SemaphoreType.DMA,
            pltpu.SemaphoreType.DMA,
            pltpu.SemaphoreType.DMA,
            pltpu.SemaphoreType.DMA,
            pltpu.SemaphoreType.DMA,
        ],
    )
    return f(z1t, ee)


def _p3_body(z2p_hbm, elr2_hbm, bnd2_hbm, sd_hbm, acc2_hbm,
             acc_sh, zbuf, rec0, rec1, sidx, didx,
             zr0, zr1, ed0, ed1, stb0, stb1, wbuf, bndv,
             lsem, gs0, gs1, gd0, gd1, ss0, ss1):
    cid = lax.axis_index("c")
    sid = lax.axis_index("s")
    wid = cid * NS + sid
    it16 = lax.iota(_i32, 16)
    unit = jnp.where(it16 == 0, 1.0, 0.0).astype(_f32)
    recs = (rec0, rec1)
    zrs = (zr0, zr1)
    eds = (ed0, ed1)
    stbs = (stb0, stb1)
    gss = (gs0, gs1)
    gds = (gd0, gd1)
    sss = (ss0, ss1)
    c0 = jnp.full((16,), 0, _i32)
    c1 = c0 + 1
    c40 = c0 + 40
    c41 = c0 + 41

    def _z(r, _):
        for j in range(4):
            zbuf[r, pl.ds(16 * j, 16)] = jnp.zeros((16,), _f32)
        return 0

    lax.fori_loop(0, NPT // 4, _z, 0)
    for t in range(4):
        pltpu.sync_copy(zbuf,
                        acc_sh.at[pl.ds(sid * NPT + t * (NPT // 4),
                                        NPT // 4)])
    pltpu.sync_copy(bnd2_hbm, bndv)
    plsc.subcore_barrier()

    base0 = wid * EPT1

    def issue_small(i, b):
        pltpu.async_copy(sd_hbm.at[pl.ds(base0 + i * K, K)], recs[b], lsem)

    def wait_small(i, b):
        pltpu.make_async_copy(sd_hbm.at[pl.ds(base0 + i * K, K)],
                              recs[b], lsem).wait()

    def stage_a(b, d):
        for g in range(8):
            rows = it16 + (16 * g)
            sl = pl.ds(16 * g, 16)
            sidx[b, sl] = plsc.load_gather(recs[b], [rows, c0])
            didx[d, sl] = plsc.load_gather(recs[b], [rows, c1])
        pltpu.async_copy(z2p_hbm.at[sidx.at[b]], zrs[b], gss[b])
        pltpu.async_copy(elr2_hbm.at[didx.at[d]], eds[b], gds[b])

    def wait_gathers(b, d):
        pltpu.make_async_copy(z2p_hbm.at[sidx.at[b]], zrs[b], gss[b]).wait()
        pltpu.make_async_copy(elr2_hbm.at[didx.at[d]], eds[b],
                              gds[b]).wait()

    def compute(i, b):
        base = base0 + i * K
        bnd = bndv[0]
        for g in range(8):
            rows = it16 + (16 * g)
            el = plsc.load_gather(zrs[b], [rows, c40])
            er = plsc.load_gather(eds[b], [rows, c1])
            ev = el + er
            ev = jnp.where(ev >= 0, ev, 0.2 * ev)
            ee = jnp.exp(ev - bnd)
            valid = (base + 16 * g + it16) < E
            wbuf[pl.ds(16 * g, 16)] = jnp.where(valid, ee, 0.0)

        def mbody(g, _):
            ge = jnp.full((16,), 16, _i32) * g
            for l in range(16):
                e = 16 * g + l
                w = plsc.load_gather(wbuf, [ge + l])
                for j in range(3):
                    sj = pl.ds(16 * j, 16)
                    stbs[b][e, sj] = zrs[b][e, sj] * w
                stbs[b][e, pl.ds(48, 16)] = unit * w
            return 0

        lax.fori_loop(0, K // 16, mbody, 0)

    def issue_scatter(b, d):
        pltpu.async_copy(stbs[b], acc_sh.at[didx.at[d]], sss[b], add=True)

    def wait_scatter(b, d):
        pltpu.make_async_copy(stbs[b], acc_sh.at[didx.at[d]], sss[b]).wait()

    issue_small(0, 0)
    wait_small(0, 0)
    stage_a(0, 0)
    issue_small(1, 1)

    def outer(io, _):
        for u in range(4):
            i = 4 * io + u
            b = u % 2
            d = u
            b1 = (u + 1) % 2
            d1 = (u + 1) % 4

            @pl.when(i >= 2)
            def _():
                wait_scatter(b, (u + 2) % 4)

            @pl.when(i < NCH1 - 1)
            def _():
                wait_small(i + 1, b1)
                stage_a(b1, d1)

            @pl.when(i < NCH1 - 2)
            def _():
                issue_small(i + 2, b)

            wait_gathers(b, d)
            compute(i, b)
            issue_scatter(b, d)
        return 0

    lax.fori_loop(0, NCH1 // 4, outer, 0)
    wait_scatter(0, 2)
    wait_scatter(1, 3)
    plsc.subcore_barrier()
    pltpu.sync_copy(acc_sh.at[pl.ds(sid * NPT, NPT)],
                    acc2_hbm.at[cid, pl.ds(sid * NPT, NPT)])


def _p3(z2p, elr2, bnd2, srcdst):
    mesh = plsc.VectorSubcoreMesh(core_axis_name="c", subcore_axis_name="s")
    f = pl.kernel(
        _p3_body,
        out_type=jax.ShapeDtypeStruct((2, NPAD, 64), _f32),
        mesh=mesh,
        compiler_params=pltpu.CompilerParams(needs_layout_passes=False, use_tc_tiling_on_sc=False),
        scratch_types=[
            pltpu.VMEM_SHARED((NPAD, 64), _f32),
            pltpu.VMEM((NPT // 4, 64), _f32),
            pltpu.VMEM((K, 2), _i32),
            pltpu.VMEM((K, 2), _i32),
            pltpu.VMEM((2, K), _i32),
            pltpu.VMEM((4, K), _i32),
            pltpu.VMEM((K, 48), _f32),
            pltpu.VMEM((K, 48), _f32),
            pltpu.VMEM((K, 16), _f32),
            pltpu.VMEM((K, 16), _f32),
            pltpu.VMEM((K, 64), _f32),
            pltpu.VMEM((K, 64), _f32),
            pltpu.VMEM((K,), _f32),
            pltpu.VMEM((1, 16), _f32),
        ] + [pltpu.SemaphoreType.DMA] * 7,
    )
    return f(z2p, elr2, bnd2, srcdst)


def kernel(x, edge_index, W1, a_l1, a_r1, b1, W2, a_l2, a_r2, b2):
    src = edge_index[0]
    dst = edge_index[1]
    pad = jnp.zeros((EP - E,), _i32)
    srcp = jnp.concatenate([src, pad])
    dstp = jnp.concatenate([dst, pad])
    srcdst = jnp.stack([srcp, dstp], axis=1)

    eye8 = jnp.eye(HEADS, dtype=_f32)
    C_l = (a_l1[:, :, None] * eye8[:, None, :]).reshape(HEADS * HID, HEADS)
    C_r = (a_r1[:, :, None] * eye8[:, None, :]).reshape(HEADS * HID, HEADS)
    C1 = jnp.concatenate([C_l, C_r], axis=1)
    C2 = jnp.zeros((NCLS, 16), _f32).at[:, 0].set(a_l2[0]).at[:, 1].set(a_r2[0])

    z1t, elr, bnd = _tc1(x, W1, C1)
    z1t = z1t.reshape(4 * N, 2 * HID)
    ee, den = _p1(elr, bnd, srcdst)
    out1 = _p2(z1t, ee)
    out1 = out1.reshape(4, NPAD, 2 * HID)
    z2p, elr2, bnd2 = _tc2(out1, den, b1.reshape(1, -1), W2, C2)
    acc2 = _p3(z2p, elr2, bnd2, srcdst)
    return _tc3(acc2, b2.reshape(1, -1))


# speedup vs baseline: 1.2862x; 1.1564x over previous
import jax
import jax.numpy as jnp
import numpy as np
from jax import lax
from jax.experimental import pallas as pl
from jax.experimental.pallas import tpu as pltpu
from jax.experimental.pallas import tpu_sc as plsc

N = 10000
E = 320000
IN_F = 128
HID = 64
HEADS = 8
NCLS = 40

NC, NS, L = 2, 16, 16
NW = NC * NS
K = 128
EP = 327680
EPT1 = EP // NW
EPT2 = EP // NS
NCH1 = EPT1 // K
NCH2 = EPT2 // K
NPAD = 10112
NPT = NPAD // NS

NB = 400
GRID = N // NB

_f32 = jnp.float32
_i32 = jnp.int32


def _tc1_body(x_ref, w1_ref, c1_ref, z1t_ref, elr_ref, bnd_ref):
    i = pl.program_id(0)
    z = jnp.dot(x_ref[...], w1_ref[...], preferred_element_type=_f32)
    for p in range(2):
        z1t_ref[p, :, :] = z[:, 256 * p:256 * (p + 1)].astype(jnp.bfloat16)
    elr = jnp.dot(z, c1_ref[...], preferred_element_type=_f32)
    elr_ref[...] = elr
    m = jnp.max(elr, axis=0, keepdims=True)

    @pl.when(i == 0)
    def _():
        bnd_ref[...] = m

    @pl.when(i > 0)
    def _():
        bnd_ref[...] = jnp.maximum(bnd_ref[...], m)

    @pl.when(i == pl.num_programs(0) - 1)
    def _():
        b = bnd_ref[...]
        s = b[:, 0:8] + b[:, 8:16]
        s = jnp.where(s >= 0, s, 0.2 * s)
        bnd_ref[...] = jnp.concatenate([s, s], axis=1)


def _tc1(x, W1, C1):
    return pl.pallas_call(
        _tc1_body,
        grid=(GRID,),
        in_specs=[
            pl.BlockSpec((NB, IN_F), lambda i: (i, 0)),
            pl.BlockSpec((IN_F, HEADS * HID), lambda i: (0, 0)),
            pl.BlockSpec((HEADS * HID, 16), lambda i: (0, 0)),
        ],
        out_specs=[
            pl.BlockSpec((2, NB, 4 * HID), lambda i: (0, i, 0)),
            pl.BlockSpec((NB, 16), lambda i: (i, 0)),
            pl.BlockSpec((1, 16), lambda i: (0, 0)),
        ],
        out_shape=[
            jax.ShapeDtypeStruct((2, N, 4 * HID), jnp.bfloat16),
            jax.ShapeDtypeStruct((N, 16), _f32),
            jax.ShapeDtypeStruct((1, 16), _f32),
        ],
    )(x, W1, C1)


def _tc2_body(o1_ref, den_ref, b1_ref, w2_ref, c2_ref,
              z2p_ref, elr2_ref, bnd2_ref):
    i = pl.program_id(0)
    den = den_ref[0] + den_ref[1]
    cols = []
    for h in range(HEADS):
        d = den[:, h:h + 1] + 1e-9
        q = h % 4
        v = o1_ref[h // 4][:, HID * q:HID * (q + 1)].astype(_f32) \
            / d + b1_ref[:, HID * h:HID * (h + 1)]
        v = jnp.where(v > 0, v, jnp.exp(jnp.minimum(v, 0.0)) - 1.0)
        cols.append(v)
    hb = jnp.concatenate(cols, axis=1)
    z2 = jnp.dot(hb, w2_ref[...], preferred_element_type=_f32)
    elr2 = jnp.dot(z2, c2_ref[...], preferred_element_type=_f32)
    z2p_ref[...] = jnp.concatenate(
        [z2, elr2[:, 0:2], jnp.zeros((NB, 6), _f32)], axis=1)
    elr2_ref[...] = elr2
    m = jnp.max(elr2, axis=0, keepdims=True)

    @pl.when(i == 0)
    def _():
        bnd2_ref[...] = m

    @pl.when(i > 0)
    def _():
        bnd2_ref[...] = jnp.maximum(bnd2_ref[...], m)

    @pl.when(i == pl.num_programs(0) - 1)
    def _():
        b = bnd2_ref[...]
        s = b[:, 0:1] + b[:, 1:2]
        s = jnp.where(s >= 0, s, 0.2 * s)
        bnd2_ref[...] = jnp.broadcast_to(s, (1, 16))


def _tc2(out1, den, b1, W2, C2):
    return pl.pallas_call(
        _tc2_body,
        grid=(GRID,),
        in_specs=[
            pl.BlockSpec((2, NB, 4 * HID), lambda i: (0, i, 0)),
            pl.BlockSpec((2, NB, 16), lambda i: (0, i, 0)),
            pl.BlockSpec((1, HEADS * HID), lambda i: (0, 0)),
            pl.BlockSpec((HEADS * HID, NCLS), lambda i: (0, 0)),
            pl.BlockSpec((NCLS, 16), lambda i: (0, 0)),
        ],
        out_specs=[
            pl.BlockSpec((NB, 48), lambda i: (i, 0)),
            pl.BlockSpec((NB, 16), lambda i: (i, 0)),
            pl.BlockSpec((1, 16), lambda i: (0, 0)),
        ],
        out_shape=[
            jax.ShapeDtypeStruct((N, 48), _f32),
            jax.ShapeDtypeStruct((N, 16), _f32),
            jax.ShapeDtypeStruct((1, 16), _f32),
        ],
    )(out1, den, b1, W2, C2)


def _tc3_body(a2_ref, b2_ref, out_ref):
    s = a2_ref[0] + a2_ref[1]
    d = s[:, 48:49] + 1e-9
    out_ref[...] = s[:, 0:NCLS] / d + b2_ref[...]


def _tc3(acc2, b2):
    return pl.pallas_call(
        _tc3_body,
        grid=(GRID,),
        in_specs=[
            pl.BlockSpec((2, NB, 64), lambda i: (0, i, 0)),
            pl.BlockSpec((1, NCLS), lambda i: (0, 0)),
        ],
        out_specs=pl.BlockSpec((NB, NCLS), lambda i: (i, 0)),
        out_shape=jax.ShapeDtypeStruct((N, NCLS), _f32),
    )(acc2, b2)


def _p1_body(elr_hbm, bnd_hbm, sd_hbm,
             ee_hbm, den_hbm,
             den_sh, zbuf, rec0, rec1, sidx, didx,
             es0, es1, ed0, ed1, eeb0, eeb1, bndv,
             lsem, gs0, gs1, gd0, gd1, ss0, ss1, ws0, ws1):
    cid = lax.axis_index("c")
    sid = lax.axis_index("s")
    wid = cid * NS + sid
    it16 = lax.iota(_i32, 16)
    recs = (rec0, rec1)
    ess = (es0, es1)
    eds = (ed0, ed1)
    eebs = (eeb0, eeb1)
    gss = (gs0, gs1)
    gds = (gd0, gd1)
    sss = (ss0, ss1)
    wss = (ws0, ws1)
    c0 = jnp.full((16,), 0, _i32)
    c1 = c0 + 1
    c8 = c0 + 8
    c9 = c0 + 9

    def _zr(r, _):
        zbuf[r, :] = jnp.zeros((16,), _f32)
        return 0

    lax.fori_loop(0, NPT, _zr, 0)
    pltpu.sync_copy(zbuf, den_sh.at[pl.ds(sid * NPT, NPT)])
    pltpu.sync_copy(bnd_hbm, bndv)
    plsc.subcore_barrier()

    base0 = wid * EPT1

    def issue_small(i, b):
        pltpu.async_copy(sd_hbm.at[pl.ds(base0 + i * K, K)], recs[b], lsem)

    def wait_small(i, b):
        pltpu.make_async_copy(sd_hbm.at[pl.ds(base0 + i * K, K)],
                              recs[b], lsem).wait()

    def stage_a(b, d):
        for g in range(8):
            rows = it16 + (16 * g)
            sl = pl.ds(16 * g, 16)
            sidx[b, sl] = plsc.load_gather(recs[b], [rows, c0])
            didx[d, sl] = plsc.load_gather(recs[b], [rows, c1])
        pltpu.async_copy(elr_hbm.at[sidx.at[b]], ess[b], gss[b])
        pltpu.async_copy(elr_hbm.at[didx.at[d]], eds[b], gds[b])

    def wait_gathers(b, d):
        pltpu.make_async_copy(elr_hbm.at[sidx.at[b]], ess[b], gss[b]).wait()
        pltpu.make_async_copy(elr_hbm.at[didx.at[d]], eds[b], gds[b]).wait()

    def issue_out(i, b, d):
        pltpu.async_copy(eebs[b], den_sh.at[didx.at[d]], sss[b], add=True)
        pltpu.async_copy(eebs[b], ee_hbm.at[pl.ds(base0 + i * K, K)], wss[b])

    def wait_out(i, b, d):
        pltpu.make_async_copy(eebs[b], den_sh.at[didx.at[d]], sss[b]).wait()
        pltpu.make_async_copy(eebs[b], ee_hbm.at[pl.ds(base0 + i * K, K)],
                              wss[b]).wait()

    issue_small(0, 0)
    wait_small(0, 0)
    stage_a(0, 0)
    issue_small(1, 1)

    def outer(io, _):
        for u in range(4):
            i = 4 * io + u
            b = u % 2
            d = u
            b1 = (u + 1) % 2
            d1 = (u + 1) % 4

            @pl.when(i >= 2)
            def _():
                wait_out(i - 2, b, (u + 2) % 4)

            @pl.when(i < NCH1 - 1)
            def _():
                wait_small(i + 1, b1)
                stage_a(b1, d1)

            @pl.when(i < NCH1 - 2)
            def _():
                issue_small(i + 2, b)

            wait_gathers(b, d)
            _compute_p1(i, b, d, base0, bndv, it16, ess, eds, eebs,
                        sidx, didx)
            issue_out(i, b, d)
        return 0

    lax.fori_loop(0, NCH1 // 4, outer, 0)
    wait_out(NCH1 - 2, 0, 2)
    wait_out(NCH1 - 1, 1, 3)
    plsc.subcore_barrier()
    pltpu.sync_copy(den_sh.at[pl.ds(sid * NPT, NPT)],
                    den_hbm.at[cid, pl.ds(sid * NPT, NPT)])


def _compute_p1(i, b, d, base0, bndv, it16, ess, eds, eebs, sidx, didx):
    base = base0 + i * K
    c0 = jnp.full((16,), 0, _i32)
    c8 = c0 + 8
    c9 = c0 + 9
    brow = bndv[0]
    for g in range(8):
        rows = it16 + (16 * g)
        sl = pl.ds(16 * g, 16)
        valid = (base + 16 * g + it16) < E
        for h in range(HEADS):
            el = plsc.load_gather(ess[b], [rows, c0 + h])
            er = plsc.load_gather(eds[b], [rows, c8 + h])
            ev = el + er
            ev = jnp.where(ev >= 0, ev, 0.2 * ev)
            ee = jnp.exp(ev - brow[h])
            ee = jnp.where(valid, ee, 0.0)
            plsc.store_scatter(eebs[b], [rows, c0 + h], ee)
        plsc.store_scatter(eebs[b], [rows, c8],
                           plsc.bitcast(sidx[b, sl], _f32))
        plsc.store_scatter(eebs[b], [rows, c9],
                           plsc.bitcast(didx[d, sl], _f32))


def _p1(elr, bnd, srcdst):
    mesh = plsc.VectorSubcoreMesh(core_axis_name="c", subcore_axis_name="s")
    f = pl.kernel(
        _p1_body,
        out_type=(jax.ShapeDtypeStruct((EP, 16), _f32),
                  jax.ShapeDtypeStruct((2, NPAD, 16), _f32)),
        mesh=mesh,
        compiler_params=pltpu.CompilerParams(needs_layout_passes=False, use_tc_tiling_on_sc=False),
        scratch_types=[
            pltpu.VMEM_SHARED((NPAD, 16), _f32),
            pltpu.VMEM((NPT, 16), _f32),
            pltpu.VMEM((K, 2), _i32),
            pltpu.VMEM((K, 2), _i32),
            pltpu.VMEM((2, K), _i32),
            pltpu.VMEM((4, K), _i32),
            pltpu.VMEM((K, 16), _f32),
            pltpu.VMEM((K, 16), _f32),
            pltpu.VMEM((K, 16), _f32),
            pltpu.VMEM((K, 16), _f32),
            pltpu.VMEM((K, 16), _f32),
            pltpu.VMEM((K, 16), _f32),
            pltpu.VMEM((1, 16), _f32),
        ] + [pltpu.SemaphoreType.DMA] * 9,
    )
    return f(elr, bnd, srcdst)


K2 = 64
NCH2B = EPT2 // K2
_bf16 = jnp.bfloat16


def _p2_body(z1t_hbm, ee_hbm, out1_hbm,
             acc_sh, eeA0, eeA1, idxb, dbuf, wb0, wb1, wb2, wb3, zr, stb,
             lsem, gsem0, gsem1, ssem0, ssem1):
    cid = lax.axis_index("c")
    sid = lax.axis_index("s")
    it16 = lax.iota(_i32, 16)
    eeAs = (eeA0, eeA1)
    wbs = (wb0, wb1, wb2, wb3)
    gsems = (gsem0, gsem1)
    ssems = (ssem0, ssem1)
    NG = K2 // 16

    base00 = sid * EPT2
    cN = cid * N
    c8 = jnp.full((16,), 8, _i32)
    c9 = jnp.full((16,), 9, _i32)
    chead = jnp.full((16,), 4, _i32) * cid

    def _z(r, _):
        for j in range(8):
            stb[0, r, pl.ds(32 * j, 32)] = jnp.zeros((32,), _bf16)
        return 0

    lax.fori_loop(0, K2, _z, 0)
    for t in range(9):
        pltpu.sync_copy(
            stb.at[0], acc_sh.at[pl.ds(sid * NPT + t * K2, K2)])
    pltpu.sync_copy(stb.at[0, pl.ds(0, NPT - 9 * K2)],
                    acc_sh.at[pl.ds(sid * NPT + 9 * K2, NPT - 9 * K2)])
    plsc.subcore_barrier()

    def issue_small(i, sb):
        base = base00 + i * K2
        pltpu.async_copy(ee_hbm.at[pl.ds(base, K2)], eeAs[sb], lsem)

    def wait_small(i, sb):
        base = base00 + i * K2
        pltpu.make_async_copy(ee_hbm.at[pl.ds(base, K2)], eeAs[sb],
                              lsem).wait()

    def stage_a(b, d):
        for g in range(NG):
            rows = it16 + (16 * g)
            sl = pl.ds(16 * g, 16)
            for q in range(4):
                wbs[q][b, sl] = plsc.load_gather(eeAs[b], [rows, chead + q])
            sv = plsc.bitcast(
                plsc.load_gather(eeAs[b], [rows, c8]), _i32)
            idxb[b, sl] = sv + cN
            dbuf[d, sl] = plsc.bitcast(
                plsc.load_gather(eeAs[b], [rows, c9]), _i32)
        pltpu.async_copy(z1t_hbm.at[idxb.at[b]], zr.at[b], gsems[b])

    def wait_gather(b):
        pltpu.make_async_copy(z1t_hbm.at[idxb.at[b]], zr.at[b],
                              gsems[b]).wait()

    def issue_scatter(b, d):
        pltpu.async_copy(stb.at[b], acc_sh.at[dbuf.at[d]],
                         ssems[b], add=True)

    def wait_scatter(b, d):
        pltpu.make_async_copy(stb.at[b], acc_sh.at[dbuf.at[d]],
                              ssems[b]).wait()

    def compute(b):
        cb = jnp.full((16,), b, _i32)

        def mbody(g, _):
            ge = jnp.full((16,), 16, _i32) * g

            for l in range(16):
                e = 16 * g + l
                il = ge + l
                ws = [plsc.load_gather(wbs[q], [cb, il]) for q in range(4)]
                for jj in range(8):
                    zv = zr[b, e, pl.ds(32 * jj, 32)]
                    za, zb = plsc.unpack(
                        zv, format=plsc.PackFormat.INTERLEAVED)
                    w = ws[jj // 2]
                    stb[b, e, pl.ds(32 * jj, 32)] = plsc.pack(
                        za * w, zb * w,
                        format=plsc.PackFormat.INTERLEAVED)
            return 0

        lax.fori_loop(0, NG, mbody, 0)

    issue_small(0, 0)
    wait_small(0, 0)
    stage_a(0, 0)
    issue_small(1, 1)

    def outer(io, _):
        for u in range(4):
            i = 4 * io + u
            b = u % 2
            d = u
            b1 = (u + 1) % 2
            d1 = (u + 1) % 4

            @pl.when(i >= 2)
            def _():
                wait_scatter(b, (u + 2) % 4)

            @pl.when(i < NCH2B - 1)
            def _():
                wait_small(i + 1, b1)
                stage_a(b1, d1)

            @pl.when(i < NCH2B - 2)
            def _():
                issue_small(i + 2, b)

            wait_gather(b)
            compute(b)
            issue_scatter(b, d)
        return 0

    lax.fori_loop(0, NCH2B // 4, outer, 0)
    wait_scatter(0, 2)
    wait_scatter(1, 3)
    plsc.subcore_barrier()
    pltpu.sync_copy(acc_sh.at[pl.ds(sid * NPT, NPT)],
                    out1_hbm.at[cid, pl.ds(sid * NPT, NPT)])


def _p2(z1t, ee):
    mesh = plsc.VectorSubcoreMesh(core_axis_name="c", subcore_axis_name="s")
    f = pl.kernel(
        _p2_body,
        out_type=jax.ShapeDtypeStruct((2, NPAD, 4 * HID), _bf16),
        mesh=mesh,
        compiler_params=pltpu.CompilerParams(needs_layout_passes=False, use_tc_tiling_on_sc=False),
        scratch_types=[
            pltpu.VMEM_SHARED((NPAD, 4 * HID), _bf16),
            pltpu.VMEM((K2, 16), _f32),
            pltpu.VMEM((K2, 16), _f32),
            pltpu.VMEM((2, K2), _i32),
            pltpu.VMEM((4, K2), _i32),
            pltpu.VMEM((2, K2), _f32),
            pltpu.VMEM((2, K2), _f32),
            pltpu.VMEM((2, K2), _f32),
            pltpu.VMEM((2, K2), _f32),
            pltpu.VMEM((2, K2, 4 * HID), _bf16),
            pltpu.VMEM((2, K2, 4 * HID), _bf16),
            pltpu.SemaphoreType.DMA,
            pltpu.SemaphoreType.DMA,
            pltpu.SemaphoreType.DMA,
            pltpu.SemaphoreType.DMA,
            pltpu.SemaphoreType.DMA,
        ],
    )
    return f(z1t, ee)


def _p3_body(z2p_hbm, elr2_hbm, bnd2_hbm, sd_hbm, acc2_hbm,
             acc_sh, zbuf, rec0, rec1, sidx, didx,
             zr0, zr1, ed0, ed1, stb0, stb1, wbuf, bndv,
             lsem, gs0, gs1, gd0, gd1, ss0, ss1):
    cid = lax.axis_index("c")
    sid = lax.axis_index("s")
    wid = cid * NS + sid
    it16 = lax.iota(_i32, 16)
    unit = jnp.where(it16 == 0, 1.0, 0.0).astype(_f32)
    recs = (rec0, rec1)
    zrs = (zr0, zr1)
    eds = (ed0, ed1)
    stbs = (stb0, stb1)
    gss = (gs0, gs1)
    gds = (gd0, gd1)
    sss = (ss0, ss1)
    c0 = jnp.full((16,), 0, _i32)
    c1 = c0 + 1
    c40 = c0 + 40
    c41 = c0 + 41

    def _z(r, _):
        for j in range(4):
            zbuf[r, pl.ds(16 * j, 16)] = jnp.zeros((16,), _f32)
        return 0

    lax.fori_loop(0, NPT // 4, _z, 0)
    for t in range(4):
        pltpu.sync_copy(zbuf,
                        acc_sh.at[pl.ds(sid * NPT + t * (NPT // 4),
                                        NPT // 4)])
    pltpu.sync_copy(bnd2_hbm, bndv)
    plsc.subcore_barrier()

    base0 = wid * EPT1

    def issue_small(i, b):
        pltpu.async_copy(sd_hbm.at[pl.ds(base0 + i * K, K)], recs[b], lsem)

    def wait_small(i, b):
        pltpu.make_async_copy(sd_hbm.at[pl.ds(base0 + i * K, K)],
                              recs[b], lsem).wait()

    def stage_a(b, d):
        for g in range(8):
            rows = it16 + (16 * g)
            sl = pl.ds(16 * g, 16)
            sidx[b, sl] = plsc.load_gather(recs[b], [rows, c0])
            didx[d, sl] = plsc.load_gather(recs[b], [rows, c1])
        pltpu.async_copy(z2p_hbm.at[sidx.at[b]], zrs[b], gss[b])
        pltpu.async_copy(elr2_hbm.at[didx.at[d]], eds[b], gds[b])

    def wait_gathers(b, d):
        pltpu.make_async_copy(z2p_hbm.at[sidx.at[b]], zrs[b], gss[b]).wait()
        pltpu.make_async_copy(elr2_hbm.at[didx.at[d]], eds[b],
                              gds[b]).wait()

    def compute(i, b):
        base = base0 + i * K
        bnd = bndv[0]
        for g in range(8):
            rows = it16 + (16 * g)
            el = plsc.load_gather(zrs[b], [rows, c40])
            er = plsc.load_gather(eds[b], [rows, c1])
            ev = el + er
            ev = jnp.where(ev >= 0, ev, 0.2 * ev)
            ee = jnp.exp(ev - bnd)
            valid = (base + 16 * g + it16) < E
            wbuf[pl.ds(16 * g, 16)] = jnp.where(valid, ee, 0.0)

        def mbody(g, _):
            ge = jnp.full((16,), 16, _i32) * g
            for l in range(16):
                e = 16 * g + l
                w = plsc.load_gather(wbuf, [ge + l])
                for j in range(3):
                    sj = pl.ds(16 * j, 16)
                    stbs[b][e, sj] = zrs[b][e, sj] * w
                stbs[b][e, pl.ds(48, 16)] = unit * w
            return 0

        lax.fori_loop(0, K // 16, mbody, 0)

    def issue_scatter(b, d):
        pltpu.async_copy(stbs[b], acc_sh.at[didx.at[d]], sss[b], add=True)

    def wait_scatter(b, d):
        pltpu.make_async_copy(stbs[b], acc_sh.at[didx.at[d]], sss[b]).wait()

    issue_small(0, 0)
    wait_small(0, 0)
    stage_a(0, 0)
    issue_small(1, 1)

    def outer(io, _):
        for u in range(4):
            i = 4 * io + u
            b = u % 2
            d = u
            b1 = (u + 1) % 2
            d1 = (u + 1) % 4

            @pl.when(i >= 2)
            def _():
                wait_scatter(b, (u + 2) % 4)

            @pl.when(i < NCH1 - 1)
            def _():
                wait_small(i + 1, b1)
                stage_a(b1, d1)

            @pl.when(i < NCH1 - 2)
            def _():
                issue_small(i + 2, b)

            wait_gathers(b, d)
            compute(i, b)
            issue_scatter(b, d)
        return 0

    lax.fori_loop(0, NCH1 // 4, outer, 0)
    wait_scatter(0, 2)
    wait_scatter(1, 3)
    plsc.subcore_barrier()
    pltpu.sync_copy(acc_sh.at[pl.ds(sid * NPT, NPT)],
                    acc2_hbm.at[cid, pl.ds(sid * NPT, NPT)])


def _p3(z2p, elr2, bnd2, srcdst):
    mesh = plsc.VectorSubcoreMesh(core_axis_name="c", subcore_axis_name="s")
    f = pl.kernel(
        _p3_body,
        out_type=jax.ShapeDtypeStruct((2, NPAD, 64), _f32),
        mesh=mesh,
        compiler_params=pltpu.CompilerParams(needs_layout_passes=False, use_tc_tiling_on_sc=False),
        scratch_types=[
            pltpu.VMEM_SHARED((NPAD, 64), _f32),
            pltpu.VMEM((NPT // 4, 64), _f32),
            pltpu.VMEM((K, 2), _i32),
            pltpu.VMEM((K, 2), _i32),
            pltpu.VMEM((2, K), _i32),
            pltpu.VMEM((4, K), _i32),
            pltpu.VMEM((K, 48), _f32),
            pltpu.VMEM((K, 48), _f32),
            pltpu.VMEM((K, 16), _f32),
            pltpu.VMEM((K, 16), _f32),
            pltpu.VMEM((K, 64), _f32),
            pltpu.VMEM((K, 64), _f32),
            pltpu.VMEM((K,), _f32),
            pltpu.VMEM((1, 16), _f32),
        ] + [pltpu.SemaphoreType.DMA] * 7,
    )
    return f(z2p, elr2, bnd2, srcdst)


def kernel(x, edge_index, W1, a_l1, a_r1, b1, W2, a_l2, a_r2, b2):
    src = edge_index[0]
    dst = edge_index[1]
    pad = jnp.zeros((EP - E,), _i32)
    srcp = jnp.concatenate([src, pad])
    dstp = jnp.concatenate([dst, pad])
    srcdst = jnp.stack([srcp, dstp], axis=1)

    eye8 = jnp.eye(HEADS, dtype=_f32)
    C_l = (a_l1[:, :, None] * eye8[:, None, :]).reshape(HEADS * HID, HEADS)
    C_r = (a_r1[:, :, None] * eye8[:, None, :]).reshape(HEADS * HID, HEADS)
    C1 = jnp.concatenate([C_l, C_r], axis=1)
    C2 = jnp.zeros((NCLS, 16), _f32).at[:, 0].set(a_l2[0]).at[:, 1].set(a_r2[0])

    z1t, elr, bnd = _tc1(x, W1, C1)
    z1t = z1t.reshape(2 * N, 4 * HID)
    ee, den = _p1(elr, bnd, srcdst)
    out1 = _p2(z1t, ee)
    out1 = out1
    z2p, elr2, bnd2 = _tc2(out1, den, b1.reshape(1, -1), W2, C2)
    acc2 = _p3(z2p, elr2, bnd2, srcdst)
    return _tc3(acc2, b2.reshape(1, -1))
